# baseline (speedup 1.0000x reference)
import functools

import jax
import jax.numpy as jnp
from jax import lax
from jax.experimental import pallas as pl
from jax.experimental.pallas import tpu as pltpu
from jax.experimental.pallas import tpu_sc as plsc

NC = 2
NS = 16
NW = NC * NS
L = 16

B = 16384
D = 64
BPW = B // NW
CH = 128
NCH = BPW // CH

_mesh = plsc.VectorSubcoreMesh(core_axis_name="c", subcore_axis_name="s")


@functools.partial(
    pl.kernel,
    out_type=jax.ShapeDtypeStruct((B,), jnp.float32),
    mesh=_mesh,
    compiler_params=pltpu.CompilerParams(
        needs_layout_passes=False, use_tc_tiling_on_sc=False),
    scratch_types=[
        pltpu.VMEM((NCH, CH), jnp.int32),
        pltpu.VMEM((NCH, CH), jnp.int32),
        pltpu.VMEM((D, CH), jnp.float32),
        pltpu.VMEM((D, CH), jnp.float32),
        pltpu.VMEM((BPW,), jnp.float32),
        pltpu.SemaphoreType.DMA,
    ],
)
def _mf_sc(u_hbm, v_hbm, ut_hbm, it_hbm, out_hbm,
           uidx, vidx, gu, gv, outv, sem):
    wid = lax.axis_index("s") * NC + lax.axis_index("c")
    base = wid * BPW

    for c in range(NCH):
        pltpu.sync_copy(u_hbm.at[pl.ds(base + c * CH, CH)], uidx.at[c])
        pltpu.sync_copy(v_hbm.at[pl.ds(base + c * CH, CH)], vidx.at[c])

    def chunk_body(c, carry):
        copies = []
        for d in range(D):
            copies.append(pltpu.async_copy(
                ut_hbm.at[d].at[uidx.at[c]], gu.at[d], sem))
            copies.append(pltpu.async_copy(
                it_hbm.at[d].at[vidx.at[c]], gv.at[d], sem))
        for cp in copies:
            cp.wait()
        for g in range(CH // L):
            acc = gu[0, pl.ds(g * L, L)] * gv[0, pl.ds(g * L, L)]
            for d in range(1, D):
                acc = acc + gu[d, pl.ds(g * L, L)] * gv[d, pl.ds(g * L, L)]
            outv[pl.ds(c * CH + g * L, L)] = acc
        return carry

    lax.fori_loop(0, NCH, chunk_body, 0)

    pltpu.sync_copy(outv, out_hbm.at[pl.ds(base, BPW)])


def kernel(u, v, user_emb, item_emb):
    return _mf_sc(u, v, user_emb.T, item_emb.T)


# speedup vs baseline: 7.9693x; 7.9693x over previous
import functools

import jax
import jax.numpy as jnp
from jax import lax
from jax.experimental import pallas as pl
from jax.experimental.pallas import tpu as pltpu
from jax.experimental.pallas import tpu_sc as plsc

NC = 2
NS = 16
NW = NC * NS
L = 16

B = 16384
D = 64
BPW = B // NW
V = 1000001

CHW = 512
NCHK = 61
SLAB = NCHK * CHW
PADW = 128
HCAP = 4096
SENT_B = B
RING = 64

_mesh = plsc.VectorSubcoreMesh(core_axis_name="c", subcore_axis_name="s")


@functools.partial(
    pl.kernel,
    out_type=(
        jax.ShapeDtypeStruct(((B + 1) * PADW,), jnp.float32),
        jax.ShapeDtypeStruct(((B + 1) * PADW,), jnp.float32),
    ),
    mesh=_mesh,
    compiler_params=pltpu.CompilerParams(
        needs_layout_passes=False, use_tc_tiling_on_sc=True),
    scratch_types=[
        pltpu.VMEM((B,), jnp.int32),
        pltpu.VMEM((B,), jnp.int32),
        pltpu.VMEM((D, CHW), jnp.float32),
        pltpu.VMEM((HCAP,), jnp.int32),
        pltpu.VMEM((HCAP,), jnp.int32),
        pltpu.VMEM((RING, D), jnp.float32),
        pltpu.SMEM((1,), jnp.int32),
        pltpu.SemaphoreType.DMA,
        pltpu.SemaphoreType.DMA,
    ],
)
def _extract(u_hbm, v_hbm, ut_hbm, it_hbm, ut_tail_hbm, it_tail_hbm,
             gue_hbm, gve_hbm,
             u_all, v_all, stg, hit_b, hit_c, ring, nwr, sem, wsem):
    wid = lax.axis_index("s") * NC + lax.axis_index("c")
    lane = lax.iota(jnp.int32, L)
    last = wid == NW - 1

    pltpu.sync_copy(u_hbm, u_all)
    pltpu.sync_copy(v_hbm, v_all)
    nwr[0] = 0

    slab_lo = wid * SLAB
    slab_hi = jnp.where(last, V, slab_lo + SLAB)

    def scan_hits(idx_all, skip):
        def group(g, n):
            uu = idx_all[pl.ds(g * L, L)]
            m = (uu >= slab_lo) & (uu < slab_hi)
            inc = plsc.cumsum(m.astype(jnp.int32))
            pos = n + inc - 1 - skip
            ok = m & (pos >= 0) & (pos < HCAP)
            posc = jnp.clip(pos, 0, HCAP - 1)
            plsc.store_scatter(hit_b, [posc], g * L + lane, mask=ok)
            plsc.store_scatter(hit_c, [posc], uu - slab_lo, mask=ok)
            return n + inc[15]

        total = lax.fori_loop(0, B // L, group, jnp.int32(0))
        stored = jnp.clip(total - skip, 0, HCAP)
        goff = (stored // L) * L

        @pl.when(goff < HCAP)
        def _():
            keep = lane < (stored - goff)
            bb = hit_b[pl.ds(goff, L)]
            cc = hit_c[pl.ds(goff, L)]
            hit_b[pl.ds(goff, L)] = jnp.where(keep, bb, SENT_B)
            hit_c[pl.ds(goff, L)] = jnp.where(keep, cc, 0)

        return total, stored

    def emit_row(col_local, b):
        n = nwr[0]
        slot = lax.rem(n, RING)

        @pl.when(n >= RING)
        def _():
            pltpu.make_async_copy(
                gue_hbm.at[pl.ds(0, D)], ring.at[slot], wsem).wait()

        for q in range(D // L):
            ring[slot, pl.ds(q * L, L)] = plsc.load_gather(
                stg, [lane + q * L, jnp.full((L,), col_local, jnp.int32)])
        nwr[0] = n + 1
        return slot

    def run_table(idx_all, tab_hbm, tail_hbm, out_hbm):
        total, stored0 = scan_hits(idx_all, 0)
        nrounds = (total + (HCAP - 1)) // HCAP

        def process_round(stored):
            def hits_in(clo, cw, stage_w):
                def hgroup(j, x):
                    bb = hit_b[pl.ds(j * L, L)]
                    cc = hit_c[pl.ds(j * L, L)]
                    inb = ((cc >= clo) & (cc < clo + cw)).astype(jnp.int32)
                    for k in range(L):
                        @pl.when(inb[k] != 0)
                        def _():
                            slot = emit_row(cc[k] - clo, bb[k])
                            pltpu.async_copy(
                                ring.at[slot],
                                out_hbm.at[pl.ds(
                                    pl.multiple_of(bb[k] * PADW, 128), D)],
                                wsem)
                    return x

                ngroups = (stored + L - 1) // L
                lax.fori_loop(0, ngroups, hgroup, 0)

            def main_chunk(c, x):
                clo = c * CHW
                pltpu.async_copy(
                    tab_hbm.at[:, pl.ds(pl.multiple_of(slab_lo + clo, 128),
                                        CHW)],
                    stg, sem).wait()
                hits_in(clo, CHW, CHW)
                return x

            lax.fori_loop(0, NCHK, main_chunk, 0)

            @pl.when(last)
            def _():
                clo = NCHK * CHW
                pltpu.async_copy(
                    tab_hbm.at[:, pl.ds(pl.multiple_of(slab_lo + clo, 128),
                                        CHW)],
                    stg, sem).wait()
                hits_in(clo, CHW, CHW)
                pltpu.async_copy(
                    tail_hbm, stg.at[:, pl.ds(0, PADW)], sem).wait()
                hits_in((NCHK + 1) * CHW, PADW, PADW)

        process_round(stored0)

        def extra_round(r, x):
            _, stored = scan_hits(idx_all, r * HCAP)
            process_round(stored)
            return x

        lax.fori_loop(1, nrounds, extra_round, 0)

    run_table(u_all, ut_hbm, ut_tail_hbm, gue_hbm)
    run_table(v_all, it_hbm, it_tail_hbm, gve_hbm)

    def drain(i, x):
        pltpu.make_async_copy(
            gue_hbm.at[pl.ds(0, D)], ring.at[0], wsem).wait()
        return x

    lax.fori_loop(0, jnp.minimum(nwr[0], RING), drain, 0)


@functools.partial(
    pl.kernel,
    out_type=jax.ShapeDtypeStruct((B,), jnp.float32),
    mesh=_mesh,
    compiler_params=pltpu.CompilerParams(
        needs_layout_passes=False, use_tc_tiling_on_sc=True),
    scratch_types=[
        pltpu.VMEM((BPW // 2 * PADW,), jnp.float32),
        pltpu.VMEM((BPW // 2 * PADW,), jnp.float32),
        pltpu.VMEM((BPW,), jnp.float32),
        pltpu.VMEM((L * (L + 1),), jnp.float32),
    ],
)
def _dots(gue_hbm, gve_hbm, out_hbm, ue, ve, outv, pbuf):
    wid = lax.axis_index("s") * NC + lax.axis_index("c")
    base = wid * BPW
    lane = lax.iota(jnp.int32, L)
    half = BPW // 2

    for h in range(2):
        lo = pl.multiple_of((base + h * half) * PADW, 128)
        pltpu.sync_copy(gue_hbm.at[pl.ds(lo, half * PADW)], ue)
        pltpu.sync_copy(gve_hbm.at[pl.ds(lo, half * PADW)], ve)

        def group_body(g, carry):
            base_r = g * L
            for i in range(L):
                r = base_r + i
                acc = (ue[pl.ds(r * PADW, L)] * ve[pl.ds(r * PADW, L)])
                for q in range(1, D // L):
                    acc = acc + (ue[pl.ds(r * PADW + q * L, L)]
                                 * ve[pl.ds(r * PADW + q * L, L)])
                plsc.store_scatter(pbuf, [lane * (L + 1) + i], acc)
            s = pbuf[pl.ds(0, L)]
            for l in range(1, L):
                s = s + pbuf[pl.ds(l * (L + 1), L)]
            outv[pl.ds(h * half + base_r, L)] = s
            return carry

        lax.fori_loop(0, half // L, group_body, 0)

    pltpu.sync_copy(outv, out_hbm.at[pl.ds(base, BPW)])


def kernel(u, v, user_emb, item_emb):
    ut = user_emb.T
    it = item_emb.T
    tail0 = NW * SLAB + CHW
    pad = ((0, 0), (0, PADW - (V - tail0)))
    ut_tail = jnp.pad(ut[:, tail0:], pad)
    it_tail = jnp.pad(it[:, tail0:], pad)
    gue, gve = _extract(u, v, ut, it, ut_tail, it_tail)
    return _dots(gue, gve)


# speedup vs baseline: 22.0511x; 2.7670x over previous
import functools

import jax
import jax.numpy as jnp
from jax import lax
from jax.experimental import pallas as pl
from jax.experimental.pallas import tpu as pltpu
from jax.experimental.pallas import tpu_sc as plsc

NC = 2
NS = 16
NW = NC * NS
L = 16

B = 16384
D = 64
BPW = B // NW
V = 1000001

CHW = 512
NCHK = 61
SLAB = NCHK * CHW
PADW = 128
HCAP = 4096
SENT_B = B
RING = 64

_mesh = plsc.VectorSubcoreMesh(core_axis_name="c", subcore_axis_name="s")


@functools.partial(
    pl.kernel,
    out_type=(
        jax.ShapeDtypeStruct(((B + 1) * PADW,), jnp.float32),
        jax.ShapeDtypeStruct(((B + 1) * PADW,), jnp.float32),
    ),
    mesh=_mesh,
    compiler_params=pltpu.CompilerParams(
        needs_layout_passes=False, use_tc_tiling_on_sc=True),
    scratch_types=[
        pltpu.VMEM((B,), jnp.int32),
        pltpu.VMEM((2, D, CHW), jnp.float32),
        pltpu.VMEM((HCAP,), jnp.int32),
        pltpu.VMEM((HCAP,), jnp.int32),
        pltpu.VMEM((RING, D), jnp.float32),
        pltpu.SMEM((1,), jnp.int32),
        pltpu.SemaphoreType.DMA,
        pltpu.SemaphoreType.DMA,
        pltpu.SemaphoreType.DMA,
    ],
)
def _extract(u_hbm, v_hbm, ut_hbm, it_hbm, ut_tail_hbm, it_tail_hbm,
             gue_hbm, gve_hbm,
             idx_all, stg2, hit_b, hit_c, ring, nwr, sem0, sem1, wsem):
    wid = lax.axis_index("s") * NC + lax.axis_index("c")
    lane = lax.iota(jnp.int32, L)
    last = wid == NW - 1

    nwr[0] = 0

    slab_lo = wid * SLAB
    slab_hi = jnp.where(last, V, slab_lo + SLAB)

    def scan_hits(idx_all, skip):
        def group(g, n):
            uu = idx_all[pl.ds(g * L, L)]
            m = (uu >= slab_lo) & (uu < slab_hi)
            inc = plsc.cumsum(m.astype(jnp.int32))
            pos = n + inc - 1 - skip
            ok = m & (pos >= 0) & (pos < HCAP)
            posc = jnp.clip(pos, 0, HCAP - 1)
            plsc.store_scatter(hit_b, [posc], g * L + lane, mask=ok)
            plsc.store_scatter(hit_c, [posc], uu - slab_lo, mask=ok)
            return n + inc[15]

        total = lax.fori_loop(0, B // L, group, jnp.int32(0))
        stored = jnp.clip(total - skip, 0, HCAP)
        goff = (stored // L) * L

        @pl.when(goff < HCAP)
        def _():
            keep = lane < (stored - goff)
            bb = hit_b[pl.ds(goff, L)]
            cc = hit_c[pl.ds(goff, L)]
            hit_b[pl.ds(goff, L)] = jnp.where(keep, bb, SENT_B)
            hit_c[pl.ds(goff, L)] = jnp.where(keep, cc, 0)

        return total, stored

    stg_a = stg2.at[0]
    stg_b = stg2.at[1]

    def emit_row(stg, col_local, b):
        n = nwr[0]
        slot = lax.rem(n, RING)

        @pl.when(n >= RING)
        def _():
            pltpu.make_async_copy(
                gue_hbm.at[pl.ds(0, D)], ring.at[slot], wsem).wait()

        for q in range(D // L):
            ring[slot, pl.ds(q * L, L)] = plsc.load_gather(
                stg, [lane + q * L, jnp.full((L,), col_local, jnp.int32)])
        nwr[0] = n + 1
        return slot

    def run_table(src_idx_hbm, tab_hbm, tail_hbm, out_hbm):
        pltpu.sync_copy(src_idx_hbm, idx_all)
        total, _ = scan_hits(idx_all, 0)
        nrounds = (total + (HCAP - 1)) // HCAP

        NCHK2 = NCHK + 1

        def fire(c):
            pltpu.async_copy(
                tab_hbm.at[:, pl.ds(pl.multiple_of(slab_lo + c * CHW, 128),
                                    CHW)],
                stg2.at[c % 2], sem0)

        def wait(c):
            pltpu.make_async_copy(
                tab_hbm.at[:, pl.ds(pl.multiple_of(slab_lo + c * CHW, 128),
                                    CHW)],
                stg2.at[c % 2], sem0).wait()

        def all_rounds(r, x):
            @pl.when(r > 0)
            def _():
                scan_hits(idx_all, r * HCAP)

            stored = jnp.clip(total - r * HCAP, 0, HCAP)
            ngroups = (stored + L - 1) // L

            def hits_in(stg, clo, cw):
                def hgroup(j, x2):
                    bb = hit_b[pl.ds(j * L, L)]
                    cc = hit_c[pl.ds(j * L, L)]
                    inb = ((cc >= clo) & (cc < clo + cw)).astype(jnp.int32)

                    @pl.when(jnp.sum(inb) != 0)
                    def _():
                        for k in range(L):
                            @pl.when(inb[k] != 0)
                            def _():
                                slot = emit_row(stg, cc[k] - clo, bb[k])
                                pltpu.async_copy(
                                    ring.at[slot],
                                    out_hbm.at[pl.ds(
                                        pl.multiple_of(bb[k] * PADW, 128),
                                        D)],
                                    wsem)
                    return x2

                lax.fori_loop(0, ngroups, hgroup, 0)

            fire(0)

            def chunk_loop(c, x2):
                @pl.when(c < NCHK2 - 1)
                def _():
                    fire(c + 1)

                wait(c)
                hits_in(stg2.at[c % 2], c * CHW, CHW)
                return x2

            lax.fori_loop(0, NCHK2, chunk_loop, 0)

            @pl.when(last)
            def _():
                pltpu.async_copy(
                    tail_hbm, stg2.at[0].at[:, pl.ds(0, PADW)], sem0).wait()
                hits_in(stg2.at[0], (NCHK + 1) * CHW, PADW)

            return x

        lax.fori_loop(0, nrounds, all_rounds, 0)

    run_table(u_hbm, ut_hbm, ut_tail_hbm, gue_hbm)
    run_table(v_hbm, it_hbm, it_tail_hbm, gve_hbm)

    def drain(i, x):
        pltpu.make_async_copy(
            gue_hbm.at[pl.ds(0, D)], ring.at[0], wsem).wait()
        return x

    lax.fori_loop(0, jnp.minimum(nwr[0], RING), drain, 0)


@functools.partial(
    pl.kernel,
    out_type=jax.ShapeDtypeStruct((B,), jnp.float32),
    mesh=_mesh,
    compiler_params=pltpu.CompilerParams(
        needs_layout_passes=False, use_tc_tiling_on_sc=True),
    scratch_types=[
        pltpu.VMEM((BPW // 2 * PADW,), jnp.float32),
        pltpu.VMEM((BPW // 2 * PADW,), jnp.float32),
        pltpu.VMEM((BPW,), jnp.float32),
        pltpu.VMEM((L * (L + 1),), jnp.float32),
    ],
)
def _dots(gue_hbm, gve_hbm, out_hbm, ue, ve, outv, pbuf):
    wid = lax.axis_index("s") * NC + lax.axis_index("c")
    base = wid * BPW
    lane = lax.iota(jnp.int32, L)
    half = BPW // 2

    for h in range(2):
        lo = pl.multiple_of((base + h * half) * PADW, 128)
        pltpu.sync_copy(gue_hbm.at[pl.ds(lo, half * PADW)], ue)
        pltpu.sync_copy(gve_hbm.at[pl.ds(lo, half * PADW)], ve)

        def group_body(g, carry):
            base_r = g * L
            for i in range(L):
                r = base_r + i
                acc = (ue[pl.ds(r * PADW, L)] * ve[pl.ds(r * PADW, L)])
                for q in range(1, D // L):
                    acc = acc + (ue[pl.ds(r * PADW + q * L, L)]
                                 * ve[pl.ds(r * PADW + q * L, L)])
                plsc.store_scatter(pbuf, [lane * (L + 1) + i], acc)
            s = pbuf[pl.ds(0, L)]
            for l in range(1, L):
                s = s + pbuf[pl.ds(l * (L + 1), L)]
            outv[pl.ds(h * half + base_r, L)] = s
            return carry

        lax.fori_loop(0, half // L, group_body, 0)

    pltpu.sync_copy(outv, out_hbm.at[pl.ds(base, BPW)])


def kernel(u, v, user_emb, item_emb):
    ut = user_emb.T
    it = item_emb.T
    tail0 = NW * SLAB + CHW
    pad = ((0, 0), (0, PADW - (V - tail0)))
    ut_tail = jnp.pad(ut[:, tail0:], pad)
    it_tail = jnp.pad(it[:, tail0:], pad)
    gue, gve = _extract(u, v, ut, it, ut_tail, it_tail)
    return _dots(gue, gve)


# speedup vs baseline: 22.6654x; 1.0279x over previous
import functools

import jax
import jax.numpy as jnp
from jax import lax
from jax.experimental import pallas as pl
from jax.experimental.pallas import tpu as pltpu
from jax.experimental.pallas import tpu_sc as plsc

NC = 2
NS = 16
NW = NC * NS
L = 16

B = 16384
D = 64
BPW = B // NW
V = 1000001

CHW = 512
NCHK = 61
SLAB = NCHK * CHW
PADW = 128
HCAP = 4096
SENT_B = B
RING = 64

_mesh = plsc.VectorSubcoreMesh(core_axis_name="c", subcore_axis_name="s")


@functools.partial(
    pl.kernel,
    out_type=(
        jax.ShapeDtypeStruct(((B + 1) * PADW,), jnp.float32),
        jax.ShapeDtypeStruct(((B + 1) * PADW,), jnp.float32),
    ),
    mesh=_mesh,
    compiler_params=pltpu.CompilerParams(
        needs_layout_passes=False, use_tc_tiling_on_sc=True),
    scratch_types=[
        pltpu.VMEM((B,), jnp.int32),
        pltpu.VMEM((2, D, CHW), jnp.float32),
        pltpu.VMEM((HCAP,), jnp.int32),
        pltpu.VMEM((HCAP,), jnp.int32),
        pltpu.VMEM((RING, D), jnp.float32),
        pltpu.SMEM((1,), jnp.int32),
        pltpu.SemaphoreType.DMA,
        pltpu.SemaphoreType.DMA,
        pltpu.SemaphoreType.DMA,
    ],
)
def _extract(u_hbm, v_hbm, ut_hbm, it_hbm, ut_tail_hbm, it_tail_hbm,
             gue_hbm, gve_hbm,
             idx_all, stg2, hit_b, hit_c, ring, nwr, sem0, sem1, wsem):
    wid = lax.axis_index("s") * NC + lax.axis_index("c")
    lane = lax.iota(jnp.int32, L)
    last = wid == NW - 1

    nwr[0] = 0

    slab_lo = wid * SLAB
    slab_hi = jnp.where(last, V, slab_lo + SLAB)

    def scan_hits(idx_all, skip):
        def group(g, n):
            uu = idx_all[pl.ds(g * L, L)]
            m = (uu >= slab_lo) & (uu < slab_hi)
            pc = plsc.all_reduce_population_count(m)
            inc = plsc.cumsum(m.astype(jnp.int32))
            pos = n + inc - 1 - skip
            ok = m & (pos >= 0) & (pos < HCAP)
            posc = jnp.clip(pos, 0, HCAP - 1)
            plsc.store_scatter(hit_b, [posc], g * L + lane, mask=ok)
            plsc.store_scatter(hit_c, [posc], uu - slab_lo, mask=ok)
            return n + pc[0]

        total = lax.fori_loop(0, B // L, group, jnp.int32(0))
        stored = jnp.clip(total - skip, 0, HCAP)
        goff = (stored // L) * L

        @pl.when(goff < HCAP)
        def _():
            keep = lane < (stored - goff)
            bb = hit_b[pl.ds(goff, L)]
            cc = hit_c[pl.ds(goff, L)]
            hit_b[pl.ds(goff, L)] = jnp.where(keep, bb, SENT_B)
            hit_c[pl.ds(goff, L)] = jnp.where(keep, cc, 0)

        return total, stored

    stg_a = stg2.at[0]
    stg_b = stg2.at[1]

    def emit_row(stg, col_local, b):
        n = nwr[0]
        slot = lax.rem(n, RING)

        @pl.when(n >= RING)
        def _():
            pltpu.make_async_copy(
                gue_hbm.at[pl.ds(0, D)], ring.at[slot], wsem).wait()

        for q in range(D // L):
            ring[slot, pl.ds(q * L, L)] = plsc.load_gather(
                stg, [lane + q * L, jnp.full((L,), col_local, jnp.int32)])
        nwr[0] = n + 1
        return slot

    def run_table(src_idx_hbm, tab_hbm, tail_hbm, out_hbm):
        pltpu.sync_copy(src_idx_hbm, idx_all)
        total, _ = scan_hits(idx_all, 0)
        nrounds = (total + (HCAP - 1)) // HCAP

        NCHK2 = NCHK + 1

        def fire(c):
            pltpu.async_copy(
                tab_hbm.at[:, pl.ds(pl.multiple_of(slab_lo + c * CHW, 128),
                                    CHW)],
                stg2.at[c % 2], sem0)

        def wait(c):
            pltpu.make_async_copy(
                tab_hbm.at[:, pl.ds(pl.multiple_of(slab_lo + c * CHW, 128),
                                    CHW)],
                stg2.at[c % 2], sem0).wait()

        def all_rounds(r, x):
            @pl.when(r > 0)
            def _():
                scan_hits(idx_all, r * HCAP)

            stored = jnp.clip(total - r * HCAP, 0, HCAP)
            ngroups = (stored + L - 1) // L

            def hits_in(stg, clo, cw):
                def hgroup(j, x2):
                    bb = hit_b[pl.ds(j * L, L)]
                    cc = hit_c[pl.ds(j * L, L)]
                    m = (cc >= clo) & (cc < clo + cw)
                    inb = m.astype(jnp.int32)

                    @pl.when(plsc.all_reduce_population_count(m)[0] != 0)
                    def _():
                        for k in range(L):
                            @pl.when(inb[k] != 0)
                            def _():
                                slot = emit_row(stg, cc[k] - clo, bb[k])
                                pltpu.async_copy(
                                    ring.at[slot],
                                    out_hbm.at[pl.ds(
                                        pl.multiple_of(bb[k] * PADW, 128),
                                        D)],
                                    wsem)
                    return x2

                lax.fori_loop(0, ngroups, hgroup, 0)

            fire(0)

            def chunk_loop(c, x2):
                @pl.when(c < NCHK2 - 1)
                def _():
                    fire(c + 1)

                wait(c)
                hits_in(stg2.at[c % 2], c * CHW, CHW)
                return x2

            lax.fori_loop(0, NCHK2, chunk_loop, 0)

            @pl.when(last)
            def _():
                pltpu.async_copy(
                    tail_hbm, stg2.at[0].at[:, pl.ds(0, PADW)], sem0).wait()
                hits_in(stg2.at[0], (NCHK + 1) * CHW, PADW)

            return x

        lax.fori_loop(0, nrounds, all_rounds, 0)

    run_table(u_hbm, ut_hbm, ut_tail_hbm, gue_hbm)
    run_table(v_hbm, it_hbm, it_tail_hbm, gve_hbm)

    def drain(i, x):
        pltpu.make_async_copy(
            gue_hbm.at[pl.ds(0, D)], ring.at[0], wsem).wait()
        return x

    lax.fori_loop(0, jnp.minimum(nwr[0], RING), drain, 0)


@functools.partial(
    pl.kernel,
    out_type=jax.ShapeDtypeStruct((B,), jnp.float32),
    mesh=_mesh,
    compiler_params=pltpu.CompilerParams(
        needs_layout_passes=False, use_tc_tiling_on_sc=True),
    scratch_types=[
        pltpu.VMEM((BPW // 2 * PADW,), jnp.float32),
        pltpu.VMEM((BPW // 2 * PADW,), jnp.float32),
        pltpu.VMEM((BPW,), jnp.float32),
        pltpu.VMEM((L * (L + 1),), jnp.float32),
    ],
)
def _dots(gue_hbm, gve_hbm, out_hbm, ue, ve, outv, pbuf):
    wid = lax.axis_index("s") * NC + lax.axis_index("c")
    base = wid * BPW
    lane = lax.iota(jnp.int32, L)
    half = BPW // 2

    for h in range(2):
        lo = pl.multiple_of((base + h * half) * PADW, 128)
        pltpu.sync_copy(gue_hbm.at[pl.ds(lo, half * PADW)], ue)
        pltpu.sync_copy(gve_hbm.at[pl.ds(lo, half * PADW)], ve)

        def group_body(g, carry):
            base_r = g * L
            for i in range(L):
                r = base_r + i
                acc = (ue[pl.ds(r * PADW, L)] * ve[pl.ds(r * PADW, L)])
                for q in range(1, D // L):
                    acc = acc + (ue[pl.ds(r * PADW + q * L, L)]
                                 * ve[pl.ds(r * PADW + q * L, L)])
                plsc.store_scatter(pbuf, [lane * (L + 1) + i], acc)
            s = pbuf[pl.ds(0, L)]
            for l in range(1, L):
                s = s + pbuf[pl.ds(l * (L + 1), L)]
            outv[pl.ds(h * half + base_r, L)] = s
            return carry

        lax.fori_loop(0, half // L, group_body, 0)

    pltpu.sync_copy(outv, out_hbm.at[pl.ds(base, BPW)])


def kernel(u, v, user_emb, item_emb):
    ut = user_emb.T
    it = item_emb.T
    tail0 = NW * SLAB + CHW
    pad = ((0, 0), (0, PADW - (V - tail0)))
    ut_tail = jnp.pad(ut[:, tail0:], pad)
    it_tail = jnp.pad(it[:, tail0:], pad)
    gue, gve = _extract(u, v, ut, it, ut_tail, it_tail)
    return _dots(gue, gve)


# speedup vs baseline: 36.4731x; 1.6092x over previous
import functools

import jax
import jax.numpy as jnp
from jax import lax
from jax.experimental import pallas as pl
from jax.experimental.pallas import tpu as pltpu
from jax.experimental.pallas import tpu_sc as plsc

NC = 2
NS = 16
NW = NC * NS
L = 16

B = 16384
D = 64
BPW = B // NW
V = 1000001

CHW = 512
NCHK = 61
SLAB = NCHK * CHW
PADW = 128
HCAP = 4096
SENT_B = B
RING = 64
NBKT = 64
SENT_C = (NBKT - 1) * CHW
SRTCAP = HCAP + NBKT * L

_mesh = plsc.VectorSubcoreMesh(core_axis_name="c", subcore_axis_name="s")


@functools.partial(
    pl.kernel,
    out_type=(
        jax.ShapeDtypeStruct(((B + 1) * PADW,), jnp.float32),
        jax.ShapeDtypeStruct(((B + 1) * PADW,), jnp.float32),
    ),
    mesh=_mesh,
    compiler_params=pltpu.CompilerParams(
        needs_layout_passes=False, use_tc_tiling_on_sc=True),
    scratch_types=[
        pltpu.VMEM((B,), jnp.int32),
        pltpu.VMEM((2, D, CHW), jnp.float32),
        pltpu.VMEM((HCAP,), jnp.int32),
        pltpu.VMEM((HCAP,), jnp.int32),
        pltpu.VMEM((SRTCAP,), jnp.int32),
        pltpu.VMEM((SRTCAP,), jnp.int32),
        pltpu.VMEM((NBKT,), jnp.int32),
        pltpu.VMEM((RING, D), jnp.float32),
        pltpu.SMEM((1 + 2 * NBKT,), jnp.int32),
        pltpu.SemaphoreType.DMA,
        pltpu.SemaphoreType.DMA,
        pltpu.SemaphoreType.DMA,
    ],
)
def _extract(u_hbm, v_hbm, ut_hbm, it_hbm, ut_tail_hbm, it_tail_hbm,
             gue_hbm, gve_hbm,
             idx_all, stg2, hit_b, hit_c, srt_b, srt_c, cnts, ring, nwr,
             sem0, sem1, wsem):
    wid = lax.axis_index("s") * NC + lax.axis_index("c")
    lane = lax.iota(jnp.int32, L)
    last = wid == NW - 1

    nwr[0] = 0

    slab_lo = wid * SLAB
    slab_hi = jnp.where(last, V, slab_lo + SLAB)

    def scan_hits(idx_all, skip):
        def group(g, n):
            uu = idx_all[pl.ds(g * L, L)]
            m = (uu >= slab_lo) & (uu < slab_hi)
            pc = plsc.all_reduce_population_count(m)
            inc = plsc.cumsum(m.astype(jnp.int32))
            pos = n + inc - 1 - skip
            ok = m & (pos >= 0) & (pos < HCAP)
            posc = jnp.clip(pos, 0, HCAP - 1)
            plsc.store_scatter(hit_b, [posc], g * L + lane, mask=ok)
            plsc.store_scatter(hit_c, [posc], uu - slab_lo, mask=ok)
            return n + pc[0]

        total = lax.fori_loop(0, B // L, group, jnp.int32(0))
        stored = jnp.clip(total - skip, 0, HCAP)
        goff = (stored // L) * L

        @pl.when(goff < HCAP)
        def _():
            keep = lane < (stored - goff)
            bb = hit_b[pl.ds(goff, L)]
            cc = hit_c[pl.ds(goff, L)]
            hit_b[pl.ds(goff, L)] = jnp.where(keep, bb, SENT_B)
            hit_c[pl.ds(goff, L)] = jnp.where(keep, cc, SENT_C)

        return total, stored

    def bucket_sort(stored):
        ngroups = (stored + L - 1) // L
        for t in range(NBKT // L):
            cnts[pl.ds(t * L, L)] = jnp.zeros((L,), jnp.int32)

        def hist(j, x):
            ids = lax.shift_right_logical(hit_c[pl.ds(j * L, L)], 9)
            for k in range(L):
                goff = lax.shift_left(
                    lax.shift_right_logical(ids[k], 4), 4)
                oh = (lane == (ids[k] & 15)).astype(jnp.int32)
                cnts[pl.ds(goff, L)] = cnts[pl.ds(goff, L)] + oh
            return x

        lax.fori_loop(0, ngroups, hist, 0)

        carry = jnp.int32(0)
        bases = []
        for t in range(NBKT // L):
            cv = cnts[pl.ds(t * L, L)]
            gv = lax.shift_right_logical(cv + (L - 1), 4)
            inc = plsc.cumsum(gv)
            base = inc - gv + carry
            carry = carry + inc[15]
            bases.append((base, gv))
            for k in range(L):
                nwr[1 + t * L + k] = base[k]
                nwr[1 + NBKT + t * L + k] = gv[k]

        def fill(g, x):
            srt_b[pl.ds(g * L, L)] = jnp.full((L,), SENT_B, jnp.int32)
            srt_c[pl.ds(g * L, L)] = jnp.full((L,), SENT_C, jnp.int32)
            return x

        lax.fori_loop(0, carry, fill, 0)

        for t in range(NBKT // L):
            cnts[pl.ds(t * L, L)] = bases[t][0] * L

        def reorder(j, x):
            bb = hit_b[pl.ds(j * L, L)]
            cc = hit_c[pl.ds(j * L, L)]
            ids = lax.shift_right_logical(cc, 9)
            for k in range(L):
                goff = lax.shift_left(
                    lax.shift_right_logical(ids[k], 4), 4)
                pv = cnts[pl.ds(goff, L)]
                oh = (lane == (ids[k] & 15)).astype(jnp.int32)
                pos = jnp.sum(pv * oh)
                one = lane < 1
                plsc.store_scatter(
                    srt_b, [jnp.full((L,), pos, jnp.int32)],
                    jnp.full((L,), bb[k], jnp.int32), mask=one)
                plsc.store_scatter(
                    srt_c, [jnp.full((L,), pos, jnp.int32)],
                    jnp.full((L,), cc[k], jnp.int32), mask=one)
                cnts[pl.ds(goff, L)] = pv + oh
            return x

        lax.fori_loop(0, ngroups, reorder, 0)

    stg_a = stg2.at[0]
    stg_b = stg2.at[1]

    def emit_row(stg, col_local, b):
        n = nwr[0]
        slot = lax.rem(n, RING)

        @pl.when(n >= RING)
        def _():
            pltpu.make_async_copy(
                gue_hbm.at[pl.ds(0, D)], ring.at[slot], wsem).wait()

        for q in range(D // L):
            ring[slot, pl.ds(q * L, L)] = plsc.load_gather(
                stg, [lane + q * L, jnp.full((L,), col_local, jnp.int32)])
        nwr[0] = n + 1
        return slot

    def run_table(src_idx_hbm, tab_hbm, tail_hbm, out_hbm):
        pltpu.sync_copy(src_idx_hbm, idx_all)
        total, _ = scan_hits(idx_all, 0)
        nrounds = (total + (HCAP - 1)) // HCAP

        NCHK2 = NCHK + 1

        def fire(c):
            pltpu.async_copy(
                tab_hbm.at[:, pl.ds(pl.multiple_of(slab_lo + c * CHW, 128),
                                    CHW)],
                stg2.at[c % 2], sem0)

        def wait(c):
            pltpu.make_async_copy(
                tab_hbm.at[:, pl.ds(pl.multiple_of(slab_lo + c * CHW, 128),
                                    CHW)],
                stg2.at[c % 2], sem0).wait()

        def all_rounds(r, x):
            @pl.when(r > 0)
            def _():
                scan_hits(idx_all, r * HCAP)

            stored = jnp.clip(total - r * HCAP, 0, HCAP)
            bucket_sort(stored)

            def hits_in(stg, clo, cw, bkt):
                g0 = nwr[1 + bkt]
                ng = nwr[1 + NBKT + bkt]

                def hgroup(jj, x2):
                    j = g0 + jj
                    bb = srt_b[pl.ds(j * L, L)]
                    cc = srt_c[pl.ds(j * L, L)]
                    m = (cc >= clo) & (cc < clo + cw)
                    inb = m.astype(jnp.int32)
                    for k in range(L):
                        @pl.when(inb[k] != 0)
                        def _():
                            slot = emit_row(stg, cc[k] - clo, bb[k])
                            pltpu.async_copy(
                                ring.at[slot],
                                out_hbm.at[pl.ds(
                                    pl.multiple_of(bb[k] * PADW, 128),
                                    D)],
                                wsem)
                    return x2

                lax.fori_loop(0, ng, hgroup, 0)

            fire(0)

            def chunk_loop(c, x2):
                @pl.when(c < NCHK2 - 1)
                def _():
                    fire(c + 1)

                wait(c)
                hits_in(stg2.at[c % 2], c * CHW, CHW, c)
                return x2

            lax.fori_loop(0, NCHK2, chunk_loop, 0)

            @pl.when(last)
            def _():
                pltpu.async_copy(
                    tail_hbm, stg2.at[0].at[:, pl.ds(0, PADW)], sem0).wait()
                hits_in(stg2.at[0], (NCHK + 1) * CHW, PADW, NBKT - 2)

            return x

        lax.fori_loop(0, nrounds, all_rounds, 0)

    run_table(u_hbm, ut_hbm, ut_tail_hbm, gue_hbm)
    run_table(v_hbm, it_hbm, it_tail_hbm, gve_hbm)

    def drain(i, x):
        pltpu.make_async_copy(
            gue_hbm.at[pl.ds(0, D)], ring.at[0], wsem).wait()
        return x

    lax.fori_loop(0, jnp.minimum(nwr[0], RING), drain, 0)


@functools.partial(
    pl.kernel,
    out_type=jax.ShapeDtypeStruct((B,), jnp.float32),
    mesh=_mesh,
    compiler_params=pltpu.CompilerParams(
        needs_layout_passes=False, use_tc_tiling_on_sc=True),
    scratch_types=[
        pltpu.VMEM((BPW // 2 * PADW,), jnp.float32),
        pltpu.VMEM((BPW // 2 * PADW,), jnp.float32),
        pltpu.VMEM((BPW,), jnp.float32),
        pltpu.VMEM((L * (L + 1),), jnp.float32),
    ],
)
def _dots(gue_hbm, gve_hbm, out_hbm, ue, ve, outv, pbuf):
    wid = lax.axis_index("s") * NC + lax.axis_index("c")
    base = wid * BPW
    lane = lax.iota(jnp.int32, L)
    half = BPW // 2

    for h in range(2):
        lo = pl.multiple_of((base + h * half) * PADW, 128)
        pltpu.sync_copy(gue_hbm.at[pl.ds(lo, half * PADW)], ue)
        pltpu.sync_copy(gve_hbm.at[pl.ds(lo, half * PADW)], ve)

        def group_body(g, carry):
            base_r = g * L
            for i in range(L):
                r = base_r + i
                acc = (ue[pl.ds(r * PADW, L)] * ve[pl.ds(r * PADW, L)])
                for q in range(1, D // L):
                    acc = acc + (ue[pl.ds(r * PADW + q * L, L)]
                                 * ve[pl.ds(r * PADW + q * L, L)])
                plsc.store_scatter(pbuf, [lane * (L + 1) + i], acc)
            s = pbuf[pl.ds(0, L)]
            for l in range(1, L):
                s = s + pbuf[pl.ds(l * (L + 1), L)]
            outv[pl.ds(h * half + base_r, L)] = s
            return carry

        lax.fori_loop(0, half // L, group_body, 0)

    pltpu.sync_copy(outv, out_hbm.at[pl.ds(base, BPW)])


def kernel(u, v, user_emb, item_emb):
    ut = user_emb.T
    it = item_emb.T
    tail0 = NW * SLAB + CHW
    pad = ((0, 0), (0, PADW - (V - tail0)))
    ut_tail = jnp.pad(ut[:, tail0:], pad)
    it_tail = jnp.pad(it[:, tail0:], pad)
    gue, gve = _extract(u, v, ut, it, ut_tail, it_tail)
    return _dots(gue, gve)


# speedup vs baseline: 38.8628x; 1.0655x over previous
import functools

import jax
import jax.numpy as jnp
from jax import lax
from jax.experimental import pallas as pl
from jax.experimental.pallas import tpu as pltpu
from jax.experimental.pallas import tpu_sc as plsc

NC = 2
NS = 16
NW = NC * NS
L = 16

B = 16384
D = 64
BPW = B // NW
V = 1000001

CHW = 256
NCHK = 122
SLAB = NCHK * CHW
PADW = 128
HCAP = 4096
SENT_B = B
RING = 64
NBKT = 128
SENT_C = (NBKT - 1) * CHW
SRTCAP = HCAP + NBKT * L

_mesh = plsc.VectorSubcoreMesh(core_axis_name="c", subcore_axis_name="s")


@functools.partial(
    pl.kernel,
    out_type=(
        jax.ShapeDtypeStruct(((B + 1) * PADW,), jnp.float32),
        jax.ShapeDtypeStruct(((B + 1) * PADW,), jnp.float32),
    ),
    mesh=_mesh,
    compiler_params=pltpu.CompilerParams(
        needs_layout_passes=False, use_tc_tiling_on_sc=True),
    scratch_types=[
        pltpu.VMEM((B,), jnp.int32),
        pltpu.VMEM((4, D, CHW), jnp.float32),
        pltpu.VMEM((HCAP,), jnp.int32),
        pltpu.VMEM((HCAP,), jnp.int32),
        pltpu.VMEM((SRTCAP,), jnp.int32),
        pltpu.VMEM((SRTCAP,), jnp.int32),
        pltpu.VMEM((NBKT,), jnp.int32),
        pltpu.VMEM((RING, D), jnp.float32),
        pltpu.SMEM((1 + 2 * NBKT,), jnp.int32),
        pltpu.SemaphoreType.DMA,
        pltpu.SemaphoreType.DMA,
        pltpu.SemaphoreType.DMA,
    ],
)
def _extract(u_hbm, v_hbm, ut_hbm, it_hbm, ut_tail_hbm, it_tail_hbm,
             gue_hbm, gve_hbm,
             idx_all, stg2, hit_b, hit_c, srt_b, srt_c, cnts, ring, nwr,
             sem0, sem1, wsem):
    wid = lax.axis_index("s") * NC + lax.axis_index("c")
    lane = lax.iota(jnp.int32, L)
    last = wid == NW - 1

    nwr[0] = 0

    slab_lo = wid * SLAB
    slab_hi = jnp.where(last, V, slab_lo + SLAB)

    def scan_hits(idx_all, skip):
        def group(g, n):
            uu = idx_all[pl.ds(g * L, L)]
            m = (uu >= slab_lo) & (uu < slab_hi)
            pc = plsc.all_reduce_population_count(m)
            inc = plsc.cumsum(m.astype(jnp.int32))
            pos = n + inc - 1 - skip
            ok = m & (pos >= 0) & (pos < HCAP)
            posc = jnp.clip(pos, 0, HCAP - 1)
            plsc.store_scatter(hit_b, [posc], g * L + lane, mask=ok)
            plsc.store_scatter(hit_c, [posc], uu - slab_lo, mask=ok)
            return n + pc[0]

        total = lax.fori_loop(0, B // L, group, jnp.int32(0))
        stored = jnp.clip(total - skip, 0, HCAP)
        goff = (stored // L) * L

        @pl.when(goff < HCAP)
        def _():
            keep = lane < (stored - goff)
            bb = hit_b[pl.ds(goff, L)]
            cc = hit_c[pl.ds(goff, L)]
            hit_b[pl.ds(goff, L)] = jnp.where(keep, bb, SENT_B)
            hit_c[pl.ds(goff, L)] = jnp.where(keep, cc, SENT_C)

        return total, stored

    def bucket_sort(stored):
        ngroups = (stored + L - 1) // L
        for t in range(NBKT // L):
            cnts[pl.ds(t * L, L)] = jnp.zeros((L,), jnp.int32)

        def hist(j, x):
            ids = lax.shift_right_logical(hit_c[pl.ds(j * L, L)], 8)
            for k in range(L):
                goff = lax.shift_left(
                    lax.shift_right_logical(ids[k], 4), 4)
                oh = (lane == (ids[k] & 15)).astype(jnp.int32)
                cnts[pl.ds(goff, L)] = cnts[pl.ds(goff, L)] + oh
            return x

        lax.fori_loop(0, ngroups, hist, 0)

        carry = jnp.int32(0)
        bases = []
        for t in range(NBKT // L):
            cv = cnts[pl.ds(t * L, L)]
            gv = lax.shift_right_logical(cv + (L - 1), 4)
            inc = plsc.cumsum(gv)
            base = inc - gv + carry
            carry = carry + inc[15]
            bases.append((base, gv))
            for k in range(L):
                nwr[1 + t * L + k] = base[k]
                nwr[1 + NBKT + t * L + k] = gv[k]

        def fill(g, x):
            srt_b[pl.ds(g * L, L)] = jnp.full((L,), SENT_B, jnp.int32)
            srt_c[pl.ds(g * L, L)] = jnp.full((L,), SENT_C, jnp.int32)
            return x

        lax.fori_loop(0, carry, fill, 0)

        for t in range(NBKT // L):
            cnts[pl.ds(t * L, L)] = bases[t][0] * L

        def reorder(j, x):
            bb = hit_b[pl.ds(j * L, L)]
            cc = hit_c[pl.ds(j * L, L)]
            ids = lax.shift_right_logical(cc, 8)
            for k in range(L):
                goff = lax.shift_left(
                    lax.shift_right_logical(ids[k], 4), 4)
                pv = cnts[pl.ds(goff, L)]
                oh = (lane == (ids[k] & 15)).astype(jnp.int32)
                pos = jnp.sum(pv * oh)
                one = lane < 1
                plsc.store_scatter(
                    srt_b, [jnp.full((L,), pos, jnp.int32)],
                    jnp.full((L,), bb[k], jnp.int32), mask=one)
                plsc.store_scatter(
                    srt_c, [jnp.full((L,), pos, jnp.int32)],
                    jnp.full((L,), cc[k], jnp.int32), mask=one)
                cnts[pl.ds(goff, L)] = pv + oh
            return x

        lax.fori_loop(0, ngroups, reorder, 0)

    stg_a = stg2.at[0]
    stg_b = stg2.at[1]

    def emit_row(stg, col_local, b):
        n = nwr[0]
        slot = lax.rem(n, RING)

        @pl.when(n >= RING)
        def _():
            pltpu.make_async_copy(
                gue_hbm.at[pl.ds(0, D)], ring.at[slot], wsem).wait()

        for q in range(D // L):
            ring[slot, pl.ds(q * L, L)] = plsc.load_gather(
                stg, [lane + q * L, jnp.full((L,), col_local, jnp.int32)])
        nwr[0] = n + 1
        return slot

    def run_table(src_idx_hbm, tab_hbm, tail_hbm, out_hbm):
        pltpu.sync_copy(src_idx_hbm, idx_all)
        total, _ = scan_hits(idx_all, 0)
        nrounds = (total + (HCAP - 1)) // HCAP

        NCHK2 = NCHK + 2

        def fire(c):
            pltpu.async_copy(
                tab_hbm.at[:, pl.ds(pl.multiple_of(slab_lo + c * CHW, 128),
                                    CHW)],
                stg2.at[c % 4], sem0)

        def wait(c):
            pltpu.make_async_copy(
                tab_hbm.at[:, pl.ds(pl.multiple_of(slab_lo + c * CHW, 128),
                                    CHW)],
                stg2.at[c % 4], sem0).wait()

        def all_rounds(r, x):
            @pl.when(r > 0)
            def _():
                scan_hits(idx_all, r * HCAP)

            stored = jnp.clip(total - r * HCAP, 0, HCAP)
            bucket_sort(stored)

            def hits_in(stg, clo, cw, bkt):
                g0 = nwr[1 + bkt]
                ng = nwr[1 + NBKT + bkt]

                def hgroup(jj, x2):
                    j = g0 + jj
                    bb = srt_b[pl.ds(j * L, L)]
                    cc = srt_c[pl.ds(j * L, L)]
                    m = (cc >= clo) & (cc < clo + cw)
                    inb = m.astype(jnp.int32)
                    for k in range(L):
                        @pl.when(inb[k] != 0)
                        def _():
                            slot = emit_row(stg, cc[k] - clo, bb[k])
                            pltpu.async_copy(
                                ring.at[slot],
                                out_hbm.at[pl.ds(
                                    pl.multiple_of(bb[k] * PADW, 128),
                                    D)],
                                wsem)
                    return x2

                lax.fori_loop(0, ng, hgroup, 0)

            fire(0)
            fire(1)
            fire(2)

            def chunk_loop(c, x2):
                @pl.when(c < NCHK2 - 3)
                def _():
                    fire(c + 3)

                wait(c)
                hits_in(stg2.at[c % 4], c * CHW, CHW, c)
                return x2

            lax.fori_loop(0, NCHK2, chunk_loop, 0)

            @pl.when(last)
            def _():
                pltpu.async_copy(
                    tail_hbm, stg2.at[0].at[:, pl.ds(0, PADW)], sem0).wait()
                hits_in(stg2.at[0], (NCHK + 2) * CHW, PADW, (NCHK + 2) * CHW // CHW)

            return x

        lax.fori_loop(0, nrounds, all_rounds, 0)

    run_table(u_hbm, ut_hbm, ut_tail_hbm, gue_hbm)
    run_table(v_hbm, it_hbm, it_tail_hbm, gve_hbm)

    def drain(i, x):
        pltpu.make_async_copy(
            gue_hbm.at[pl.ds(0, D)], ring.at[0], wsem).wait()
        return x

    lax.fori_loop(0, jnp.minimum(nwr[0], RING), drain, 0)


@functools.partial(
    pl.kernel,
    out_type=jax.ShapeDtypeStruct((B,), jnp.float32),
    mesh=_mesh,
    compiler_params=pltpu.CompilerParams(
        needs_layout_passes=False, use_tc_tiling_on_sc=True),
    scratch_types=[
        pltpu.VMEM((BPW // 2 * PADW,), jnp.float32),
        pltpu.VMEM((BPW // 2 * PADW,), jnp.float32),
        pltpu.VMEM((BPW,), jnp.float32),
        pltpu.VMEM((L * (L + 1),), jnp.float32),
    ],
)
def _dots(gue_hbm, gve_hbm, out_hbm, ue, ve, outv, pbuf):
    wid = lax.axis_index("s") * NC + lax.axis_index("c")
    base = wid * BPW
    lane = lax.iota(jnp.int32, L)
    half = BPW // 2

    for h in range(2):
        lo = pl.multiple_of((base + h * half) * PADW, 128)
        pltpu.sync_copy(gue_hbm.at[pl.ds(lo, half * PADW)], ue)
        pltpu.sync_copy(gve_hbm.at[pl.ds(lo, half * PADW)], ve)

        def group_body(g, carry):
            base_r = g * L
            for i in range(L):
                r = base_r + i
                acc = (ue[pl.ds(r * PADW, L)] * ve[pl.ds(r * PADW, L)])
                for q in range(1, D // L):
                    acc = acc + (ue[pl.ds(r * PADW + q * L, L)]
                                 * ve[pl.ds(r * PADW + q * L, L)])
                plsc.store_scatter(pbuf, [lane * (L + 1) + i], acc)
            s = pbuf[pl.ds(0, L)]
            for l in range(1, L):
                s = s + pbuf[pl.ds(l * (L + 1), L)]
            outv[pl.ds(h * half + base_r, L)] = s
            return carry

        lax.fori_loop(0, half // L, group_body, 0)

    pltpu.sync_copy(outv, out_hbm.at[pl.ds(base, BPW)])


def kernel(u, v, user_emb, item_emb):
    ut = user_emb.T
    it = item_emb.T
    tail0 = NW * SLAB + 2 * CHW
    pad = ((0, 0), (0, PADW - (V - tail0)))
    ut_tail = jnp.pad(ut[:, tail0:], pad)
    it_tail = jnp.pad(it[:, tail0:], pad)
    gue, gve = _extract(u, v, ut, it, ut_tail, it_tail)
    return _dots(gue, gve)
